# baseline (speedup 1.0000x reference)
import jax
import jax.numpy as jnp
from jax import lax
from jax.experimental import pallas as pl
from jax.experimental.pallas import tpu as pltpu
from jax.experimental.pallas import tpu_sc as plsc

N = 320000
D = 128
S = 10000
NC = 2
NS = 16
HALF = D // NC
BLK = 128
NBLK = N // BLK
KB = 1
CH = KB * BLK
NBUF = 8
PREF = 7
NCHUNK = NBLK // KB
CH_PER_TILE = NCHUNK // NS
CH_REM = NCHUNK - NS * CH_PER_TILE
IDS_ROWS = (CH_PER_TILE + 1) * KB
ROWS_OUT = S // NS


def _sc_body(x_hbm, b2d_hbm, z_hbm, out_hbm, acc, data_v, ids_all, dsem,
             isem, ssem):
    c = lax.axis_index("c")
    s = lax.axis_index("s")
    col0 = c * HALF

    first = s * CH_PER_TILE + jnp.minimum(s, CH_REM)
    n_ch = jnp.where(s < CH_REM, CH_PER_TILE + 1, CH_PER_TILE)
    blk0 = first * KB

    def issue(g, b):
        pltpu.async_copy(
            x_hbm.at[pl.ds(g * CH, CH), pl.ds(col0, HALF)],
            data_v.at[b], dsem.at[b])

    for p in range(PREF):
        issue(first + p, p)
    ids_start = jnp.minimum(blk0, NBLK - IDS_ROWS)
    off = blk0 - ids_start
    pltpu.async_copy(b2d_hbm.at[pl.ds(ids_start, IDS_ROWS)], ids_all, isem)

    r0 = s * ROWS_OUT
    pltpu.sync_copy(z_hbm.at[pl.ds(r0, ROWS_OUT)], acc.at[pl.ds(r0, ROWS_OUT)])
    pltpu.make_async_copy(
        b2d_hbm.at[pl.ds(0, IDS_ROWS)], ids_all, isem).wait()
    plsc.subcore_barrier()

    def scatter(b, i):
        for j in range(KB):
            pltpu.async_copy(
                data_v.at[b, pl.ds(j * BLK, BLK)],
                acc.at[ids_all.at[off + i * KB + j]], ssem.at[b], add=True)

    def drain_scatters(b):
        for j in range(KB):
            pltpu.make_async_copy(
                data_v.at[b, pl.ds(j * BLK, BLK)],
                acc.at[ids_all.at[0]], ssem.at[b]).wait()

    def body(i, carry):
        b = lax.rem(i, NBUF)
        nxt = lax.rem(i + PREF, NBUF)
        pltpu.make_async_copy(
            x_hbm.at[pl.ds(0, CH), pl.ds(col0, HALF)],
            data_v.at[b], dsem.at[b]).wait()

        @pl.when(i >= NBUF - PREF)
        def _():
            drain_scatters(nxt)

        @pl.when(i + PREF < n_ch)
        def _():
            issue(first + i + PREF, nxt)

        scatter(b, i)
        return carry

    lax.fori_loop(0, n_ch, body, 0)
    for t in range(NBUF - PREF):
        drain_scatters(lax.rem(n_ch - 1 - t + NBUF, NBUF))
    plsc.subcore_barrier()

    pltpu.sync_copy(
        acc.at[pl.ds(r0, ROWS_OUT)],
        out_hbm.at[pl.ds(r0, ROWS_OUT), pl.ds(col0, HALF)])


def kernel(x, batch):
    batch = batch.astype(jnp.int32)
    b2d = batch.reshape(NBLK, BLK)
    zeros = jnp.zeros((S, HALF), jnp.float32)
    mesh = plsc.VectorSubcoreMesh(core_axis_name="c", subcore_axis_name="s")
    f = pl.kernel(
        _sc_body,
        out_type=jax.ShapeDtypeStruct((S, D), jnp.float32),
        mesh=mesh,
        scratch_types=[
            pltpu.VMEM_SHARED((S, HALF), jnp.float32),
            pltpu.VMEM((NBUF, CH, HALF), jnp.float32),
            pltpu.VMEM((IDS_ROWS, BLK), jnp.int32),
            pltpu.SemaphoreType.DMA((NBUF,)),
            pltpu.SemaphoreType.DMA,
            pltpu.SemaphoreType.DMA((NBUF,)),
        ],
        compiler_params=pltpu.CompilerParams(use_tc_tiling_on_sc=False),
    )
    return f(x, b2d, zeros)


# speedup vs baseline: 1.0371x; 1.0371x over previous
import jax
import jax.numpy as jnp
from jax import lax
from jax.experimental import pallas as pl
from jax.experimental.pallas import tpu as pltpu
from jax.experimental.pallas import tpu_sc as plsc

N = 320000
D = 128
S = 10000
NC = 2
NS = 16
HALF = D // NC
BLK = 128
NBLK = N // BLK
KB = 1
CH = KB * BLK
NBUF = 8
PREF = 6
NCHUNK = NBLK // KB
CH_PER_TILE = NCHUNK // NS
CH_REM = NCHUNK - NS * CH_PER_TILE
IDS_ROWS = (CH_PER_TILE + 1) * KB
ROWS_OUT = S // NS


def _sc_body(x_hbm, b2d_hbm, z_hbm, out_hbm, acc, data_v, ids_all, dsem,
             isem, ssem):
    c = lax.axis_index("c")
    s = lax.axis_index("s")
    col0 = c * HALF

    first = s * CH_PER_TILE + jnp.minimum(s, CH_REM)
    n_ch = jnp.where(s < CH_REM, CH_PER_TILE + 1, CH_PER_TILE)
    blk0 = first * KB

    def issue(g, b):
        pltpu.async_copy(
            x_hbm.at[pl.ds(g * CH, CH), pl.ds(col0, HALF)],
            data_v.at[b], dsem.at[b])

    for p in range(PREF):
        issue(first + p, p)
    ids_start = jnp.minimum(blk0, NBLK - IDS_ROWS)
    off = blk0 - ids_start
    pltpu.async_copy(b2d_hbm.at[pl.ds(ids_start, IDS_ROWS)], ids_all, isem)

    r0 = s * ROWS_OUT
    pltpu.sync_copy(z_hbm.at[pl.ds(r0, ROWS_OUT)], acc.at[pl.ds(r0, ROWS_OUT)])
    pltpu.make_async_copy(
        b2d_hbm.at[pl.ds(0, IDS_ROWS)], ids_all, isem).wait()
    plsc.subcore_barrier()

    def scatter(b, i):
        for j in range(KB):
            pltpu.async_copy(
                data_v.at[b, pl.ds(j * BLK, BLK)],
                acc.at[ids_all.at[off + i * KB + j]], ssem.at[b], add=True)

    def drain_scatters(b):
        for j in range(KB):
            pltpu.make_async_copy(
                data_v.at[b, pl.ds(j * BLK, BLK)],
                acc.at[ids_all.at[0]], ssem.at[b]).wait()

    def body(i, carry):
        b = lax.rem(i, NBUF)
        nxt = lax.rem(i + PREF, NBUF)
        pltpu.make_async_copy(
            x_hbm.at[pl.ds(0, CH), pl.ds(col0, HALF)],
            data_v.at[b], dsem.at[b]).wait()

        @pl.when(i >= NBUF - PREF)
        def _():
            drain_scatters(nxt)

        @pl.when(i + PREF < n_ch)
        def _():
            issue(first + i + PREF, nxt)

        scatter(b, i)
        return carry

    lax.fori_loop(0, n_ch, body, 0)
    for t in range(NBUF - PREF):
        drain_scatters(lax.rem(n_ch - 1 - t + NBUF, NBUF))
    plsc.subcore_barrier()

    pltpu.sync_copy(
        acc.at[pl.ds(r0, ROWS_OUT)],
        out_hbm.at[pl.ds(r0, ROWS_OUT), pl.ds(col0, HALF)])


def kernel(x, batch):
    batch = batch.astype(jnp.int32)
    b2d = batch.reshape(NBLK, BLK)
    zeros = jnp.zeros((S, HALF), jnp.float32)
    mesh = plsc.VectorSubcoreMesh(core_axis_name="c", subcore_axis_name="s")
    f = pl.kernel(
        _sc_body,
        out_type=jax.ShapeDtypeStruct((S, D), jnp.float32),
        mesh=mesh,
        scratch_types=[
            pltpu.VMEM_SHARED((S, HALF), jnp.float32),
            pltpu.VMEM((NBUF, CH, HALF), jnp.float32),
            pltpu.VMEM((IDS_ROWS, BLK), jnp.int32),
            pltpu.SemaphoreType.DMA((NBUF,)),
            pltpu.SemaphoreType.DMA,
            pltpu.SemaphoreType.DMA((NBUF,)),
        ],
        compiler_params=pltpu.CompilerParams(use_tc_tiling_on_sc=False),
    )
    return f(x, b2d, zeros)
